# baseline (speedup 1.0000x reference)
import functools
import math

import jax
import jax.numpy as jnp
import numpy as np
from jax.experimental import pallas as pl

EMBED = 1024
K = 32
_SCALE = 1.0 / math.sqrt(float(EMBED))
_NT = (((1,), (1,)), ((), ()))
_NN = (((1,), (0,)), ((), ()))


def _qkv_kernel(x_ref, wq_ref, wk_ref, wv_ref, b_ref,
                q_ref, k_ref, v_ref):
    x = x_ref[0].astype(jnp.bfloat16)
    d = x.shape[1]
    b = b_ref[...]
    q = jax.lax.dot_general(x, wq_ref[...].astype(jnp.bfloat16), _NT,
                            preferred_element_type=jnp.float32)
    k = jax.lax.dot_general(x, wk_ref[...].astype(jnp.bfloat16), _NT,
                            preferred_element_type=jnp.float32)
    v = jax.lax.dot_general(x, wv_ref[...].astype(jnp.bfloat16), _NT,
                            preferred_element_type=jnp.float32)
    q_ref[0] = (q + b[:, 0:d]).astype(jnp.bfloat16)
    k_ref[0] = (k + b[:, d:2 * d]).astype(jnp.bfloat16)
    v_ref[0] = (v + b[:, 2 * d:3 * d]).astype(jnp.bfloat16)


def _attn_kernel(q_ref, k_ref, v_ref, wo_ref, bo_ref, o_ref):
    s = jax.lax.dot_general(q_ref[0], k_ref[0], _NT,
                            preferred_element_type=jnp.float32)
    s = s * _SCALE

    bits = jax.lax.bitcast_convert_type(s, jnp.int32)
    key = jnp.where(bits >= 0, bits, bits ^ jnp.int32(0x7FFFFFFF))

    nneg = jnp.sum((key >= 0).astype(jnp.int32), axis=1, keepdims=True)
    t = jnp.where(nneg >= K, jnp.int32(0), jnp.int32(-2147483648))
    for bit in range(30, -1, -1):
        cand = t | jnp.int32(1 << bit)
        cnt = jnp.sum((key >= cand).astype(jnp.int32), axis=1, keepdims=True)
        t = jnp.where(cnt >= K, cand, t)

    mask = key >= t
    m = jnp.max(s, axis=1, keepdims=True)
    p = jnp.where(mask, jnp.exp(s - m), 0.0)
    z = jnp.sum(p, axis=1, keepdims=True)
    w = (p / z).astype(jnp.bfloat16)

    out = jax.lax.dot_general(w, v_ref[0], _NN,
                              preferred_element_type=jnp.float32)
    res = jax.lax.dot_general(out.astype(jnp.bfloat16),
                              wo_ref[...].astype(jnp.bfloat16), _NT,
                              preferred_element_type=jnp.float32)
    o_ref[0] = res + bo_ref[...]


@functools.partial(jax.jit, static_argnames=("blk_qkv", "blk"))
def _run(x, Wq, bq, Wk, bk, Wv, bv, Wo, bo, blk_qkv=512, blk=256):
    B, S, D = x.shape
    bqkv = jnp.concatenate([bq, bk, bv])[None, :]

    shape_sd = jax.ShapeDtypeStruct((B, S, D), jnp.bfloat16)
    wspec = pl.BlockSpec((D, D), lambda b, i: (0, 0))
    q, k, v = pl.pallas_call(
        _qkv_kernel,
        grid=(B, S // blk_qkv),
        in_specs=[
            pl.BlockSpec((1, blk_qkv, D), lambda b, i: (b, i, 0)),
            wspec, wspec, wspec,
            pl.BlockSpec((1, 3 * D), lambda b, i: (0, 0)),
        ],
        out_specs=(
            pl.BlockSpec((1, blk_qkv, D), lambda b, i: (b, i, 0)),
            pl.BlockSpec((1, blk_qkv, D), lambda b, i: (b, i, 0)),
            pl.BlockSpec((1, blk_qkv, D), lambda b, i: (b, i, 0)),
        ),
        out_shape=(shape_sd, shape_sd, shape_sd),
    )(x, Wq, Wk, Wv, bqkv)

    result = pl.pallas_call(
        _attn_kernel,
        grid=(B, S // blk),
        in_specs=[
            pl.BlockSpec((1, blk, D), lambda b, i: (b, i, 0)),
            pl.BlockSpec((1, S, D), lambda b, i: (b, 0, 0)),
            pl.BlockSpec((1, S, D), lambda b, i: (b, 0, 0)),
            pl.BlockSpec((D, D), lambda b, i: (0, 0)),
            pl.BlockSpec((1, D), lambda b, i: (0, 0)),
        ],
        out_specs=pl.BlockSpec((1, blk, D), lambda b, i: (b, i, 0)),
        out_shape=jax.ShapeDtypeStruct((B, S, D), jnp.float32),
    )(q, k, v, Wo, bo[None, :])
    return result


def _run_b(x, Wq, bq, Wk, bk, Wv, bv, Wo, bo):
    return _run(x, Wq, bq, Wk, bk, Wv, bv, Wo, bo)


def kernel(x, Wq, bq, Wk, bk, Wv, bv, Wo, bo):
    devs = jax.devices()
    B = x.shape[0]
    if len(devs) >= 2 and B % 2 == 0:
        mesh = jax.sharding.Mesh(np.array(devs[:2]), ("b",))
        pb = jax.sharding.PartitionSpec("b")
        pr = jax.sharding.PartitionSpec()
        shard_fn = getattr(jax, "shard_map", None)
        if shard_fn is None:
            from jax.experimental.shard_map import shard_map as shard_fn
        f = shard_fn(_run_b, mesh=mesh,
                     in_specs=(pb, pr, pr, pr, pr, pr, pr, pr, pr),
                     out_specs=pb, check_vma=False)
        return f(x, Wq, bq, Wk, bk, Wv, bv, Wo, bo)
    return _run(x, Wq, bq, Wk, bk, Wv, bv, Wo, bo)


# speedup vs baseline: 2.8694x; 2.8694x over previous
import functools
import math

import jax
import jax.numpy as jnp
import numpy as np
from jax.experimental import pallas as pl

EMBED = 1024
K = 32
_SCALE = 1.0 / math.sqrt(float(EMBED))
_NT = (((1,), (1,)), ((), ()))
_NN = (((1,), (0,)), ((), ()))


def _qkv_kernel(x_ref, wq_ref, wk_ref, wv_ref, b_ref,
                q_ref, k_ref, v_ref):
    x = x_ref[0].astype(jnp.bfloat16)
    d = x.shape[1]
    b = b_ref[...]
    q = jax.lax.dot_general(x, wq_ref[...].astype(jnp.bfloat16), _NT,
                            preferred_element_type=jnp.float32)
    k = jax.lax.dot_general(x, wk_ref[...].astype(jnp.bfloat16), _NT,
                            preferred_element_type=jnp.float32)
    v = jax.lax.dot_general(x, wv_ref[...].astype(jnp.bfloat16), _NT,
                            preferred_element_type=jnp.float32)
    q_ref[0] = (q + b[:, 0:d]).astype(jnp.bfloat16)
    k_ref[0] = (k + b[:, d:2 * d]).astype(jnp.bfloat16)
    v_ref[0] = (v + b[:, 2 * d:3 * d]).astype(jnp.bfloat16)


def _attn_kernel(q_ref, k_ref, v_ref, wo_ref, bo_ref, o_ref):
    s = jax.lax.dot_general(q_ref[0], k_ref[0], _NT,
                            preferred_element_type=jnp.float32)
    s = s * _SCALE

    bits = jax.lax.bitcast_convert_type(s, jnp.int32)
    key = jnp.where(bits >= 0, bits, bits ^ jnp.int32(0x7FFFFFFF))

    nneg = jnp.sum((key >= 0).astype(jnp.int32), axis=1, keepdims=True)
    t = jnp.where(nneg >= K, jnp.int32(0), jnp.int32(-2147483648))
    for bit in range(30, -1, -1):
        cand = t | jnp.int32(1 << bit)
        cnt = jnp.sum((key >= cand).astype(jnp.int32), axis=1, keepdims=True)
        t = jnp.where(cnt >= K, cand, t)

    mask = key >= t
    m = jnp.max(s, axis=1, keepdims=True)
    p = jnp.where(mask, jnp.exp(s - m), 0.0)
    z = jnp.sum(p, axis=1, keepdims=True)
    w = (p / z).astype(jnp.bfloat16)

    out = jax.lax.dot_general(w, v_ref[0], _NN,
                              preferred_element_type=jnp.float32)
    res = jax.lax.dot_general(out.astype(jnp.bfloat16),
                              wo_ref[...].astype(jnp.bfloat16), _NT,
                              preferred_element_type=jnp.float32)
    o_ref[0] = res + bo_ref[...]


@functools.partial(jax.jit, static_argnames=("blk_qkv", "blk"))
def _run(x, Wq, bq, Wk, bk, Wv, bv, Wo, bo, blk_qkv=512, blk=512):
    B, S, D = x.shape
    bqkv = jnp.concatenate([bq, bk, bv])[None, :]

    shape_sd = jax.ShapeDtypeStruct((B, S, D), jnp.bfloat16)
    wspec = pl.BlockSpec((D, D), lambda b, i: (0, 0))
    q, k, v = pl.pallas_call(
        _qkv_kernel,
        grid=(B, S // blk_qkv),
        in_specs=[
            pl.BlockSpec((1, blk_qkv, D), lambda b, i: (b, i, 0)),
            wspec, wspec, wspec,
            pl.BlockSpec((1, 3 * D), lambda b, i: (0, 0)),
        ],
        out_specs=(
            pl.BlockSpec((1, blk_qkv, D), lambda b, i: (b, i, 0)),
            pl.BlockSpec((1, blk_qkv, D), lambda b, i: (b, i, 0)),
            pl.BlockSpec((1, blk_qkv, D), lambda b, i: (b, i, 0)),
        ),
        out_shape=(shape_sd, shape_sd, shape_sd),
    )(x, Wq, Wk, Wv, bqkv)

    result = pl.pallas_call(
        _attn_kernel,
        grid=(B, S // blk),
        in_specs=[
            pl.BlockSpec((1, blk, D), lambda b, i: (b, i, 0)),
            pl.BlockSpec((1, S, D), lambda b, i: (b, 0, 0)),
            pl.BlockSpec((1, S, D), lambda b, i: (b, 0, 0)),
            pl.BlockSpec((D, D), lambda b, i: (0, 0)),
            pl.BlockSpec((1, D), lambda b, i: (0, 0)),
        ],
        out_specs=pl.BlockSpec((1, blk, D), lambda b, i: (b, i, 0)),
        out_shape=jax.ShapeDtypeStruct((B, S, D), jnp.float32),
    )(q, k, v, Wo, bo[None, :])
    return result


def kernel(x, Wq, bq, Wk, bk, Wv, bv, Wo, bo):
    return _run(x, Wq, bq, Wk, bk, Wv, bv, Wo, bo)


# speedup vs baseline: 2.8875x; 1.0063x over previous
import functools
import math

import jax
import jax.numpy as jnp
import numpy as np
from jax.experimental import pallas as pl

EMBED = 1024
K = 32
_SCALE = 1.0 / math.sqrt(float(EMBED))
_NT = (((1,), (1,)), ((), ()))
_NN = (((1,), (0,)), ((), ()))


def _qkv_kernel(x_ref, wq_ref, wk_ref, wv_ref, b_ref,
                q_ref, k_ref, v_ref):
    x = x_ref[0].astype(jnp.bfloat16)
    d = x.shape[1]
    b = b_ref[...]
    q = jax.lax.dot_general(x, wq_ref[...].astype(jnp.bfloat16), _NT,
                            preferred_element_type=jnp.float32)
    k = jax.lax.dot_general(x, wk_ref[...].astype(jnp.bfloat16), _NT,
                            preferred_element_type=jnp.float32)
    v = jax.lax.dot_general(x, wv_ref[...].astype(jnp.bfloat16), _NT,
                            preferred_element_type=jnp.float32)
    q_ref[0] = ((q + b[:, 0:d]) * _SCALE).astype(jnp.bfloat16)
    k_ref[0] = (k + b[:, d:2 * d]).astype(jnp.bfloat16)
    v_ref[0] = (v + b[:, 2 * d:3 * d]).astype(jnp.bfloat16)


def _attn_kernel(q_ref, k_ref, v_ref, wo_ref, bo_ref, o_ref):
    s = jax.lax.dot_general(q_ref[0], k_ref[0], _NT,
                            preferred_element_type=jnp.float32)

    bits = jax.lax.bitcast_convert_type(s, jnp.int32)
    key = jnp.where(bits >= 0, bits, bits ^ jnp.int32(0x7FFFFFFF))

    m = jnp.max(s, axis=1, keepdims=True)
    e = jnp.exp(s - m)

    nneg = jnp.sum((key >= 0).astype(jnp.int32), axis=1, keepdims=True)
    t = jnp.where(nneg >= K, jnp.int32(0), jnp.int32(-2147483648))
    for bit in range(30, -1, -1):
        cand = t | jnp.int32(1 << bit)
        cnt = jnp.sum((key >= cand).astype(jnp.int32), axis=1, keepdims=True)
        t = jnp.where(cnt >= K, cand, t)

    p = jnp.where(key >= t, e, 0.0)
    z = jnp.sum(p, axis=1, keepdims=True)
    w = (p / z).astype(jnp.bfloat16)

    out = jax.lax.dot_general(w, v_ref[0], _NN,
                              preferred_element_type=jnp.float32)
    res = jax.lax.dot_general(out.astype(jnp.bfloat16),
                              wo_ref[...].astype(jnp.bfloat16), _NT,
                              preferred_element_type=jnp.float32)
    o_ref[0] = res + bo_ref[...]


@functools.partial(jax.jit, static_argnames=("blk_qkv", "blk"))
def _run(x, Wq, bq, Wk, bk, Wv, bv, Wo, bo, blk_qkv=1024, blk=512):
    B, S, D = x.shape
    bqkv = jnp.concatenate([bq, bk, bv])[None, :]

    shape_sd = jax.ShapeDtypeStruct((B, S, D), jnp.bfloat16)
    wspec = pl.BlockSpec((D, D), lambda b, i: (0, 0))
    q, k, v = pl.pallas_call(
        _qkv_kernel,
        grid=(B, S // blk_qkv),
        in_specs=[
            pl.BlockSpec((1, blk_qkv, D), lambda b, i: (b, i, 0)),
            wspec, wspec, wspec,
            pl.BlockSpec((1, 3 * D), lambda b, i: (0, 0)),
        ],
        out_specs=(
            pl.BlockSpec((1, blk_qkv, D), lambda b, i: (b, i, 0)),
            pl.BlockSpec((1, blk_qkv, D), lambda b, i: (b, i, 0)),
            pl.BlockSpec((1, blk_qkv, D), lambda b, i: (b, i, 0)),
        ),
        out_shape=(shape_sd, shape_sd, shape_sd),
    )(x, Wq, Wk, Wv, bqkv)

    result = pl.pallas_call(
        _attn_kernel,
        grid=(B, S // blk),
        in_specs=[
            pl.BlockSpec((1, blk, D), lambda b, i: (b, i, 0)),
            pl.BlockSpec((1, S, D), lambda b, i: (b, 0, 0)),
            pl.BlockSpec((1, S, D), lambda b, i: (b, 0, 0)),
            pl.BlockSpec((D, D), lambda b, i: (0, 0)),
            pl.BlockSpec((1, D), lambda b, i: (0, 0)),
        ],
        out_specs=pl.BlockSpec((1, blk, D), lambda b, i: (b, i, 0)),
        out_shape=jax.ShapeDtypeStruct((B, S, D), jnp.float32),
    )(q, k, v, Wo, bo[None, :])
    return result


def kernel(x, Wq, bq, Wk, bk, Wv, bv, Wo, bo):
    return _run(x, Wq, bq, Wk, bk, Wv, bv, Wo, bo)
